# Initial kernel scaffold; baseline (speedup 1.0000x reference)
import jax
import jax.numpy as jnp
from jax.experimental import pallas as pl


def kernel(text, max_seq_len, letter_embedding, positional_encoding):
    raise NotImplementedError("write your pallas kernel here")


# speedup vs baseline: 3.0420x; 3.0420x over previous
import functools

import jax
import jax.numpy as jnp
from jax import lax
from jax.experimental import pallas as pl
from jax.experimental.pallas import tpu as pltpu
from jax.experimental.pallas import tpu_sc as plsc

_NC = 2
_NS = 16
_NW = _NC * _NS

_CHUNK = 128


@functools.partial(jax.jit, static_argnums=(3, 4))
def _sc_gather(idx, table, out_struct_dummy, b_per_w, d):
    del out_struct_dummy
    b = idx.shape[0]
    n_chunks = b_per_w // _CHUNK
    mesh = plsc.VectorSubcoreMesh(core_axis_name="c", subcore_axis_name="s")

    @functools.partial(
        pl.kernel,
        out_type=jax.ShapeDtypeStruct((b, d), jnp.float32),
        mesh=mesh,
        scratch_types=[
            pltpu.VMEM((b_per_w,), jnp.int32),
            pltpu.VMEM((_CHUNK, d), jnp.float32),
            pltpu.SemaphoreType.DMA,
        ],
    )
    def k(idx_hbm, table_hbm, out_hbm, idx_v, rows_v, sem):
        wid = lax.axis_index("s") * _NC + lax.axis_index("c")
        base = wid * b_per_w
        pltpu.sync_copy(idx_hbm.at[pl.ds(base, b_per_w)], idx_v)

        def body(g, _):
            gbase = g * _CHUNK
            pltpu.async_copy(
                table_hbm.at[idx_v.at[pl.ds(gbase, _CHUNK)]], rows_v, sem
            ).wait()
            pltpu.sync_copy(rows_v, out_hbm.at[pl.ds(base + gbase, _CHUNK)])
            return 0

        lax.fori_loop(0, n_chunks, body, 0)

    return k(idx, table)


def kernel(text, max_seq_len, letter_embedding, positional_encoding):
    del max_seq_len, positional_encoding
    b0, s = text.shape
    v, d = letter_embedding.shape
    b = b0 * s
    idx = text.reshape(b).astype(jnp.int32)
    b_per_w = b // _NW
    out = _sc_gather(idx, letter_embedding, None, b_per_w, d)
    return out.reshape(b0, s, d)


# speedup vs baseline: 3.0721x; 1.0099x over previous
import functools

import jax
import jax.numpy as jnp
from jax import lax
from jax.experimental import pallas as pl
from jax.experimental.pallas import tpu as pltpu
from jax.experimental.pallas import tpu_sc as plsc

_NC = 2
_NS = 16
_NW = _NC * _NS

_CHUNK = 128


@functools.partial(jax.jit, static_argnums=(2, 3))
def _sc_gather(idx, table, b_per_w, d):
    b = idx.shape[0]
    n_chunks = b_per_w // _CHUNK
    mesh = plsc.VectorSubcoreMesh(core_axis_name="c", subcore_axis_name="s")

    @functools.partial(
        pl.kernel,
        out_type=jax.ShapeDtypeStruct((b, d), jnp.float32),
        mesh=mesh,
        scratch_types=[
            pltpu.VMEM((b_per_w,), jnp.int32),
            pltpu.VMEM((_CHUNK, d), jnp.float32),
            pltpu.VMEM((_CHUNK, d), jnp.float32),
            pltpu.SemaphoreType.DMA,
            pltpu.SemaphoreType.DMA,
            pltpu.SemaphoreType.DMA,
            pltpu.SemaphoreType.DMA,
        ],
    )
    def k(idx_hbm, table_hbm, out_hbm, idx_v, rows0, rows1, g0, g1, w0, w1):
        wid = lax.axis_index("s") * _NC + lax.axis_index("c")
        base = wid * b_per_w
        rows = (rows0, rows1)
        gsem = (g0, g1)
        wsem = (w0, w1)

        pltpu.sync_copy(idx_hbm.at[pl.ds(base, b_per_w)], idx_v)

        def gather_copy(g, bi):
            return pltpu.make_async_copy(
                table_hbm.at[idx_v.at[pl.ds(g * _CHUNK, _CHUNK)]],
                rows[bi],
                gsem[bi],
            )

        def write_copy(g, bi):
            return pltpu.make_async_copy(
                rows[bi],
                out_hbm.at[pl.ds(base + g * _CHUNK, _CHUNK)],
                wsem[bi],
            )

        gather_copy(0, 0).start()

        def step(g, bi):
            @pl.when(g >= 1)
            def _():
                write_copy(g - 1, bi ^ 1).wait()

            @pl.when(g + 1 < n_chunks)
            def _():
                gather_copy(g + 1, bi ^ 1).start()

            gather_copy(g, bi).wait()
            write_copy(g, bi).start()

        def body(grp, _):
            g = grp * 2
            step(g, 0)
            step(g + 1, 1)
            return 0

        lax.fori_loop(0, n_chunks // 2, body, 0)
        write_copy(n_chunks - 1, (n_chunks - 1) % 2).wait()

    return k(idx, table)


def kernel(text, max_seq_len, letter_embedding, positional_encoding):
    del max_seq_len, positional_encoding
    b0, s = text.shape
    v, d = letter_embedding.shape
    b = b0 * s
    idx = text.reshape(b).astype(jnp.int32)
    b_per_w = b // _NW
    out = _sc_gather(idx, letter_embedding, b_per_w, d)
    return out.reshape(b0, s, d)
